# baseline (speedup 1.0000x reference)
import functools

import jax
import jax.numpy as jnp
import numpy as np
from jax import lax
from jax.experimental import pallas as pl
from jax.experimental.pallas import tpu as pltpu
from jax.experimental.pallas import tpu_sc as plsc

_B = 16
_C = 255
_G = 76
_P = _G * _G
_NA = 85
_STRIDE = 8.0
_ANCHORS = (116.0, 90.0, 156.0, 198.0, 373.0, 326.0)

_NW = 32
_PB = 16
_NBLK = _P // _PB
_NTASK = _B * _NBLK
_TASKS_PER_W = -(-_NTASK // _NW)


def _decode_body(x_hbm, out_hbm, inb, outb):
    cid = lax.axis_index("c")
    sid = lax.axis_index("s")
    wid = sid * 2 + cid

    row_iota = lax.iota(jnp.int32, _PB)

    def task(i, _):
        t = i * _NW + wid

        @pl.when(t < _NTASK)
        def _():
            b = t // _NBLK
            blk = t - b * _NBLK
            p0 = blk * _PB

            pltpu.sync_copy(x_hbm.at[b, :, pl.ds(p0, _PB)], inb)

            p_vec = p0 + row_iota
            gy = p_vec // _G
            xoff = (p_vec - gy * _G).astype(jnp.float32)
            yoff = gy.astype(jnp.float32)

            def sig(v):
                return 1.0 / (1.0 + jnp.exp(-v))

            def put(c, res):
                plsc.store_scatter(
                    outb, [row_iota, jnp.full((_PB,), c, jnp.int32)], res)

            for k in range(3):
                base = k * _NA
                put(base + 0, (sig(inb[base + 0]) + xoff) * _STRIDE)
                put(base + 1, (sig(inb[base + 1]) + yoff) * _STRIDE)
                put(base + 2, jnp.exp(inb[base + 2]) * _ANCHORS[2 * k])
                put(base + 3, jnp.exp(inb[base + 3]) * _ANCHORS[2 * k + 1])
                put(base + 4, sig(inb[base + 4]))

            def cls_rows(j, _):
                for k in range(3):
                    c = k * _NA + j
                    put(c, sig(inb[c]))
                return 0

            lax.fori_loop(5, _NA, cls_rows, 0)

            pltpu.sync_copy(outb, out_hbm.at[b, pl.ds(p0, _PB), :])

        return 0

    lax.fori_loop(0, _TASKS_PER_W, task, 0)


@jax.jit
def kernel(inputs):
    x = inputs.reshape(_B, _C, _P)
    mesh = plsc.VectorSubcoreMesh(core_axis_name="c", subcore_axis_name="s")
    decode = functools.partial(
        pl.kernel,
        mesh=mesh,
        out_type=jax.ShapeDtypeStruct((_B, _P, _C), jnp.float32),
        compiler_params=pltpu.CompilerParams(
            use_tc_tiling_on_sc=False, needs_layout_passes=False),
        scratch_types=[
            pltpu.VMEM((_C, _PB), jnp.float32),
            pltpu.VMEM((_PB, _C), jnp.float32),
        ],
    )(_decode_body)
    out = decode(x)
    return out.reshape(_B, _P * 3, _NA)


# speedup vs baseline: 1.1442x; 1.1442x over previous
import functools

import jax
import jax.numpy as jnp
from jax import lax
from jax.experimental import pallas as pl
from jax.experimental.pallas import tpu as pltpu
from jax.experimental.pallas import tpu_sc as plsc

_B = 16
_C = 255
_G = 76
_P = _G * _G
_NA = 85
_STRIDE = 8.0
_ANCHORS = (116.0, 90.0, 156.0, 198.0, 373.0, 326.0)

_NW = 32
_PB = 16
_BLK = _PB * _C
_NBLK = _P // _PB
_NTASK = _B * _NBLK
_NGRP = (_NTASK // _NW + 1) // 2 + 1


def _decode_body(x_hbm, out_hbm, inb, outb, isem0, isem1, osem0, osem1):
    cid = lax.axis_index("c")
    sid = lax.axis_index("s")
    wid = sid * 2 + cid

    row_iota = lax.iota(jnp.int32, _PB)
    flat = row_iota * _C
    isems = (isem0, isem1)
    osems = (osem0, osem1)

    def in_desc(t, k):
        b = t // _NBLK
        p0 = (t - b * _NBLK) * _PB
        return pltpu.make_async_copy(
            x_hbm.at[b, :, pl.ds(p0, _PB)], inb.at[k], isems[k])

    def out_desc(t, k):
        b = t // _NBLK
        p0 = (t - b * _NBLK) * _PB
        return pltpu.make_async_copy(
            outb.at[k], out_hbm.at[b, pl.ds(p0 * _C, _BLK)], osems[k])

    def valid(i):
        t = i * _NW + wid
        return t < _NTASK

    def compute(t, k):
        p0 = (t - (t // _NBLK) * _NBLK) * _PB
        p_vec = p0 + row_iota
        gy = p_vec // _G
        xoff = (p_vec - gy * _G).astype(jnp.float32)
        yoff = gy.astype(jnp.float32)

        src = inb.at[k]
        dst = outb.at[k]

        def sig(v):
            return 1.0 / (1.0 + jnp.exp(-v))

        def put(c, res):
            plsc.store_scatter(dst, [flat + c], res)

        for a in range(3):
            base = a * _NA
            put(base + 0, (sig(src[base + 0]) + xoff) * _STRIDE)
            put(base + 1, (sig(src[base + 1]) + yoff) * _STRIDE)
            put(base + 2, jnp.exp(src[base + 2]) * _ANCHORS[2 * a])
            put(base + 3, jnp.exp(src[base + 3]) * _ANCHORS[2 * a + 1])
            for j in range(4, _NA):
                put(base + j, sig(src[base + j]))

    in_desc(wid, 0).start()

    def group(g, _):
        for kb in range(2):
            i = 2 * g + kb
            t = i * _NW + wid

            @pl.when(valid(i + 1))
            def _():
                in_desc(t + _NW, 1 - kb).start()

            @pl.when(valid(i))
            def _():
                in_desc(t, kb).wait()

                @pl.when(i >= 2)
                def _():
                    out_desc(t - 2 * _NW, kb).wait()

                compute(t, kb)
                out_desc(t, kb).start()

        return 0

    lax.fori_loop(0, _NGRP, group, 0)

    out_desc(wid, 0).wait()
    out_desc(wid, 1).wait()


@jax.jit
def kernel(inputs):
    x = inputs.reshape(_B, _C, _P)
    mesh = plsc.VectorSubcoreMesh(core_axis_name="c", subcore_axis_name="s")
    decode = functools.partial(
        pl.kernel,
        mesh=mesh,
        out_type=jax.ShapeDtypeStruct((_B, _P * _C), jnp.float32),
        compiler_params=pltpu.CompilerParams(
            use_tc_tiling_on_sc=False, needs_layout_passes=False),
        scratch_types=[
            pltpu.VMEM((2, _C, _PB), jnp.float32),
            pltpu.VMEM((2, _BLK), jnp.float32),
            pltpu.SemaphoreType.DMA,
            pltpu.SemaphoreType.DMA,
            pltpu.SemaphoreType.DMA,
            pltpu.SemaphoreType.DMA,
        ],
    )(_decode_body)
    out = decode(x)
    return out.reshape(_B, _P * 3, _NA)


# speedup vs baseline: 1.3733x; 1.2003x over previous
import functools

import jax
import jax.numpy as jnp
from jax import lax
from jax.experimental import pallas as pl
from jax.experimental.pallas import tpu as pltpu
from jax.experimental.pallas import tpu_sc as plsc

_B = 16
_C = 255
_G = 76
_P = _G * _G
_NA = 85
_STRIDE = 8.0
_ANCHORS = (116.0, 90.0, 156.0, 198.0, 373.0, 326.0)

_NW = 32
_PB = 16
_BLK = _PB * _C
_NBLK = _P // _PB
_NTASK = _B * _NBLK
_NGRP = (_NTASK // _NW + 1) // 2 + 1


def _decode_body(x_hbm, out_hbm, inb, outb, isem0, isem1, osem0, osem1):
    cid = lax.axis_index("c")
    sid = lax.axis_index("s")
    wid = sid * 2 + cid

    row_iota = lax.iota(jnp.int32, _PB)
    flat = row_iota * _C
    isems = (isem0, isem1)
    osems = (osem0, osem1)

    def in_desc(t, k):
        b = t // _NBLK
        p0 = (t - b * _NBLK) * _PB
        return pltpu.make_async_copy(
            x_hbm.at[b, :, pl.ds(p0, _PB)], inb.at[k], isems[k])

    def out_desc(t, k):
        b = t // _NBLK
        p0 = (t - b * _NBLK) * _PB
        return pltpu.make_async_copy(
            outb.at[k], out_hbm.at[b, pl.ds(p0 * _C, _BLK)], osems[k])

    def valid(i):
        t = i * _NW + wid
        return t < _NTASK

    def compute(t, k):
        p0 = (t - (t // _NBLK) * _NBLK) * _PB
        p_vec = p0 + row_iota
        gy = p_vec // _G
        xoff = (p_vec - gy * _G).astype(jnp.float32)
        yoff = gy.astype(jnp.float32)

        src = inb.at[k]
        dst = outb.at[k]

        def sig(v):
            return 1.0 / (1.0 + jnp.exp(-v))

        def row(c):
            a, j = divmod(c, _NA)
            v = src[c]
            if j == 0:
                return (sig(v) + xoff) * _STRIDE
            if j == 1:
                return (sig(v) + yoff) * _STRIDE
            if j in (2, 3):
                return jnp.exp(v) * _ANCHORS[2 * a + (j - 2)]
            return sig(v)

        bs = 15
        for c0 in range(0, _C, bs):
            cs = range(c0, min(c0 + bs, _C))
            results = [row(c) for c in cs]
            for c, res in zip(cs, results):
                plsc.store_scatter(dst, [flat + c], res)

    in_desc(wid, 0).start()

    def group(g, _):
        for kb in range(2):
            i = 2 * g + kb
            t = i * _NW + wid

            @pl.when(valid(i + 1))
            def _():
                in_desc(t + _NW, 1 - kb).start()

            @pl.when(valid(i))
            def _():
                in_desc(t, kb).wait()

                @pl.when(i >= 2)
                def _():
                    out_desc(t - 2 * _NW, kb).wait()

                compute(t, kb)
                out_desc(t, kb).start()

        return 0

    lax.fori_loop(0, _NGRP, group, 0)

    out_desc(wid, 0).wait()
    out_desc(wid, 1).wait()


@jax.jit
def kernel(inputs):
    x = inputs.reshape(_B, _C, _P)
    mesh = plsc.VectorSubcoreMesh(core_axis_name="c", subcore_axis_name="s")
    decode = functools.partial(
        pl.kernel,
        mesh=mesh,
        out_type=jax.ShapeDtypeStruct((_B, _P * _C), jnp.float32),
        compiler_params=pltpu.CompilerParams(
            use_tc_tiling_on_sc=False, needs_layout_passes=False),
        scratch_types=[
            pltpu.VMEM((2, _C, _PB), jnp.float32),
            pltpu.VMEM((2, _BLK), jnp.float32),
            pltpu.SemaphoreType.DMA,
            pltpu.SemaphoreType.DMA,
            pltpu.SemaphoreType.DMA,
            pltpu.SemaphoreType.DMA,
        ],
    )(_decode_body)
    out = decode(x)
    return out.reshape(_B, _P * 3, _NA)
